# Initial kernel scaffold; baseline (speedup 1.0000x reference)
import jax
import jax.numpy as jnp
from jax.experimental import pallas as pl


def kernel(input):
    raise NotImplementedError("write your pallas kernel here")


# speedup vs baseline: 158.7095x; 158.7095x over previous
import functools

import jax
import jax.numpy as jnp
from jax.experimental import pallas as pl
from jax.experimental.pallas import tpu as pltpu

_R = 3
_BW = 0.1

_DELTAS = [(0, 1), (0, 2),
           (1, -2), (1, -1), (1, 0), (1, 1), (1, 2),
           (2, -2), (2, -1), (2, 0), (2, 1), (2, 2)]


def _entropy_kernel(x_ref, xn_ref, out_ref, *, H, W, C):
    Ho, Wo = H - _R + 1, W - _R + 1
    inv2h2 = 1.0 / (2.0 * _BW * _BW)
    M = _R * _R

    def fmaps(ref):
        F = {}
        for (di, dj) in _DELTAS:
            r0, rN = 0, H - di
            c0, cN = max(0, -dj), W - max(0, dj)
            d2 = None
            for c in range(C):
                a = ref[0, c, r0:rN, c0:cN]
                b = ref[0, c, r0 + di:rN + di, c0 + dj:cN + dj]
                diff = a - b
                d2 = diff * diff if d2 is None else d2 + diff * diff
            F[(di, dj)] = jnp.exp(-inv2h2 * d2)
        return F

    Ff = fmaps(x_ref)
    Fn = fmaps(xn_ref)
    G = {d: Ff[d] * Fn[d] for d in _DELTAS}

    ones = jnp.ones((Ho, Wo), jnp.float32)
    S_m = [ones] * M
    S_j = [ones] * M
    for a in range(M):
        ai, aj = divmod(a, _R)
        for b in range(a + 1, M):
            bi, bj = divmod(b, _R)
            d = (bi - ai, bj - aj)
            rs = ai
            cs = min(aj, bj)
            ef = Ff[d][rs:rs + Ho, cs:cs + Wo]
            ej = G[d][rs:rs + Ho, cs:cs + Wo]
            S_m[a] = S_m[a] + ef
            S_m[b] = S_m[b] + ef
            S_j[a] = S_j[a] + ej
            S_j[b] = S_j[b] + ej

    P_m = S_m[0]
    P_j = S_j[0]
    for k in range(1, M):
        P_m = P_m * S_m[k]
        P_j = P_j * S_j[k]

    log_norm = jnp.log(2.0 * jnp.pi * _BW * _BW)
    c_m = jnp.float32(jnp.log(float(M)) + 0.5 * C * log_norm)
    c_j = jnp.float32(jnp.log(float(M)) + C * log_norm)
    inv_m = jnp.float32(1.0 / M)
    h_m = c_m - jnp.log(P_m) * inv_m
    h_j = c_j - jnp.log(P_j) * inv_m

    pad = _R // 2
    out_ref[0, :, :, :] = jnp.zeros((2, H, W), jnp.float32)
    out_ref[0, 0, pad:pad + Ho, pad:pad + Wo] = h_m
    out_ref[0, 1, pad:pad + Ho, pad:pad + Wo] = h_j


def _run(x, N, SF, C, H, W, interpret=False):
    spec_x = pl.BlockSpec((1, C, H, W), lambda i: (i, 0, 0, 0))
    spec_xn = pl.BlockSpec(
        (1, C, H, W),
        lambda i, SF=SF: (jnp.where(i % SF == SF - 1, i, i + 1), 0, 0, 0))
    return pl.pallas_call(
        functools.partial(_entropy_kernel, H=H, W=W, C=C),
        grid=(N * SF,),
        in_specs=[spec_x, spec_xn],
        out_specs=pl.BlockSpec((1, 2, H, W), lambda i: (i, 0, 0, 0)),
        out_shape=jax.ShapeDtypeStruct((N * SF, 2, H, W), jnp.float32),
        compiler_params=pltpu.CompilerParams(
            dimension_semantics=("parallel",),
            vmem_limit_bytes=100 * 1024 * 1024,
        ),
        interpret=interpret,
    )(x, x)


def kernel(input):
    N, SF, C, H, W = input.shape
    x = input.reshape(N * SF, C, H, W)
    out = _run(x, N, SF, C, H, W)
    return out.reshape(N, SF, 2, H, W)


# speedup vs baseline: 208.2482x; 1.3121x over previous
import functools
import math

import jax
import jax.numpy as jnp
from jax.experimental import pallas as pl
from jax.experimental.pallas import tpu as pltpu

_R = 3
_BW = 0.1

_DELTAS = [(0, 1), (0, 2),
           (1, -2), (1, -1), (1, 0), (1, 1), (1, 2),
           (2, -2), (2, -1), (2, 0), (2, 1), (2, 2)]


def _roll2(x, di, dj):
    if dj:
        x = jnp.roll(x, dj, axis=1)
    if di:
        x = jnp.roll(x, di, axis=0)
    return x


def _entropy_kernel(x_ref, xn_ref, out_ref, *, H, W, C):
    Ho, Wo = H - _R + 1, W - _R + 1
    inv2h2 = 1.0 / (2.0 * _BW * _BW)
    M = _R * _R
    scale = jnp.float32(math.sqrt(inv2h2 / math.log(2.0)))

    def build_maps(ref):
        Xs = [ref[0, c] * scale for c in range(C)]
        F = {}
        for dj in (-2, -1, 0, 1, 2):
            Xd = Xs if dj == 0 else [jnp.roll(xc, -dj, axis=1) for xc in Xs]
            for (di, dj2) in _DELTAS:
                if dj2 != dj:
                    continue
                acc = None
                for c in range(C):
                    b = jnp.roll(Xd[c], -di, axis=0) if di else Xd[c]
                    d = Xs[c] - b
                    acc = d * d if acc is None else acc + d * d
                F[(di, dj)] = jnp.exp2(-acc)
        return F

    Ff = build_maps(x_ref)
    Fn = build_maps(xn_ref)

    ones = jnp.ones((H, W), jnp.float32)
    Mm = {(0, 0): ones}
    Mj = {(0, 0): ones}
    for (di, dj) in _DELTAS:
        f = Ff[(di, dj)]
        g = f * Fn[(di, dj)]
        Mm[(di, dj)] = f
        Mj[(di, dj)] = g
        Mm[(-di, -dj)] = _roll2(f, di, dj)
        Mj[(-di, -dj)] = _roll2(g, di, dj)

    def windows(Md):
        return {(r, cl): Md[(r, cl)] + Md[(r, cl + 1)] + Md[(r, cl + 2)]
                for r in range(-2, 3) for cl in (-2, -1, 0)}

    Cm = windows(Mm)
    Cj = windows(Mj)

    pm = None
    pj = None
    for ai in range(_R):
        for aj in range(_R):
            sm = Cm[(-ai, -aj)] + Cm[(1 - ai, -aj)] + Cm[(2 - ai, -aj)]
            sj = Cj[(-ai, -aj)] + Cj[(1 - ai, -aj)] + Cj[(2 - ai, -aj)]
            sm = sm[ai:ai + Ho, aj:aj + Wo]
            sj = sj[ai:ai + Ho, aj:aj + Wo]
            pm = sm if pm is None else pm * sm
            pj = sj if pj is None else pj * sj

    ln2 = math.log(2.0)
    log_norm = math.log(2.0 * math.pi * _BW * _BW)
    c_m = jnp.float32(math.log(float(M)) + 0.5 * C * log_norm)
    c_j = jnp.float32(math.log(float(M)) + C * log_norm)
    h_m = c_m - jnp.log2(pm) * jnp.float32(ln2 / M)
    h_j = c_j - jnp.log2(pj) * jnp.float32(ln2 / M)

    pad = _R // 2
    out_ref[0, :, :, :] = jnp.zeros((2, H, W), jnp.float32)
    out_ref[0, 0, pad:pad + Ho, pad:pad + Wo] = h_m
    out_ref[0, 1, pad:pad + Ho, pad:pad + Wo] = h_j


def _run(x, N, SF, C, H, W, interpret=False):
    spec_x = pl.BlockSpec((1, C, H, W), lambda i: (i, 0, 0, 0))
    spec_xn = pl.BlockSpec(
        (1, C, H, W),
        lambda i, SF=SF: (jnp.where(i % SF == SF - 1, i, i + 1), 0, 0, 0))
    return pl.pallas_call(
        functools.partial(_entropy_kernel, H=H, W=W, C=C),
        grid=(N * SF,),
        in_specs=[spec_x, spec_xn],
        out_specs=pl.BlockSpec((1, 2, H, W), lambda i: (i, 0, 0, 0)),
        out_shape=jax.ShapeDtypeStruct((N * SF, 2, H, W), jnp.float32),
        compiler_params=pltpu.CompilerParams(
            dimension_semantics=("parallel",),
            vmem_limit_bytes=100 * 1024 * 1024,
        ),
        interpret=interpret,
    )(x, x)


def kernel(input):
    N, SF, C, H, W = input.shape
    x = input.reshape(N * SF, C, H, W)
    out = _run(x, N, SF, C, H, W)
    return out.reshape(N, SF, 2, H, W)


# speedup vs baseline: 232.7192x; 1.1175x over previous
import functools
import math

import jax
import jax.numpy as jnp
from jax.experimental import pallas as pl
from jax.experimental.pallas import tpu as pltpu

_R = 3
_BW = 0.1

_DELTAS = [(0, 1), (0, 2),
           (1, -2), (1, -1), (1, 0), (1, 1), (1, 2),
           (2, -2), (2, -1), (2, 0), (2, 1), (2, 2)]


def _entropy_kernel(x_ref, xn_ref, out_ref, *, H, W, C):
    Ho, Wo = H - _R + 1, W - _R + 1
    inv2h2 = 1.0 / (2.0 * _BW * _BW)
    M = _R * _R
    scale = jnp.float32(math.sqrt(inv2h2 / math.log(2.0)))

    def build_maps(ref):
        Xs = [ref[0, c] * scale for c in range(C)]
        Xd = {0: Xs}
        for dj in (-2, -1, 1, 2):
            Xd[dj] = [jnp.roll(xc, -dj, axis=1) for xc in Xs]
        Xr = {0: Xs}
        for di in (1, 2):
            Xr[di] = [jnp.roll(xc, di, axis=0) for xc in Xs]
        Ft = {}
        for (di, dj) in _DELTAS:
            acc = None
            for c in range(C):
                d = Xr[di][c] - Xd[dj][c]
                acc = d * d if acc is None else acc + d * d
            Ft[(di, dj)] = jnp.exp2(-acc)
        return Ft

    Ftf = build_maps(x_ref)
    Ftn = build_maps(xn_ref)
    Gt = {d: Ftf[d] * Ftn[d] for d in _DELTAS}

    def point_sums(T):
        Cw = {}
        m01, m02 = T[(0, 1)], T[(0, 2)]
        m0m1 = jnp.roll(m01, 1, axis=1)
        m0m2 = jnp.roll(m02, 2, axis=1)
        Cw[(0, -2)] = (m0m2 + m0m1) + 1.0
        Cw[(0, -1)] = (m0m1 + m01) + 1.0
        Cw[(0, 0)] = (m01 + m02) + 1.0
        for di in (1, 2):
            TL = {0: T[(di, 0)]}
            for cp in (-2, -1, 1, 2):
                TL[cp] = jnp.roll(T[(di, cp)], cp, axis=1)
            for cl in (-2, -1, 0):
                Cw[(-di, cl)] = TL[-cl] + TL[-(cl + 1)] + TL[-(cl + 2)]
                Cw[(di, cl)] = T[(di, cl)] + T[(di, cl + 1)] + T[(di, cl + 2)]
        return Cw

    def entropy(T, const):
        Cw = point_sums(T)
        for di in (1, 2):
            for cl in (-2, -1, 0):
                Cw[(di, cl)] = jnp.roll(Cw[(di, cl)], -di, axis=0)
        p = None
        for aj in range(_R):
            cl = -aj
            u = Cw[(-1, cl)] + Cw[(0, cl)]
            v = Cw[(0, cl)] + Cw[(1, cl)]
            s0 = v + Cw[(2, cl)]
            s1 = u + Cw[(1, cl)]
            s2 = Cw[(-2, cl)] + u
            q = s0[0:Ho, :] * s1[1:1 + Ho, :] * s2[2:2 + Ho, :]
            qs = q[:, aj:aj + Wo]
            p = qs if p is None else p * qs
        return jnp.float32(const) - jnp.log2(p) * jnp.float32(math.log(2.0) / M)

    log_norm = math.log(2.0 * math.pi * _BW * _BW)
    h_m = entropy(Ftf, math.log(float(M)) + 0.5 * C * log_norm)
    h_j = entropy(Gt, math.log(float(M)) + C * log_norm)

    pad = _R // 2
    out_ref[0, :, :, :] = jnp.zeros((2, H, W), jnp.float32)
    out_ref[0, 0, pad:pad + Ho, pad:pad + Wo] = h_m
    out_ref[0, 1, pad:pad + Ho, pad:pad + Wo] = h_j


def _run(x, N, SF, C, H, W, interpret=False):
    spec_x = pl.BlockSpec((1, C, H, W), lambda i: (i, 0, 0, 0))
    spec_xn = pl.BlockSpec(
        (1, C, H, W),
        lambda i, SF=SF: (jnp.where(i % SF == SF - 1, i, i + 1), 0, 0, 0))
    return pl.pallas_call(
        functools.partial(_entropy_kernel, H=H, W=W, C=C),
        grid=(N * SF,),
        in_specs=[spec_x, spec_xn],
        out_specs=pl.BlockSpec((1, 2, H, W), lambda i: (i, 0, 0, 0)),
        out_shape=jax.ShapeDtypeStruct((N * SF, 2, H, W), jnp.float32),
        compiler_params=pltpu.CompilerParams(
            dimension_semantics=("parallel",),
            vmem_limit_bytes=100 * 1024 * 1024,
        ),
        interpret=interpret,
    )(x, x)


def kernel(input):
    N, SF, C, H, W = input.shape
    x = input.reshape(N * SF, C, H, W)
    out = _run(x, N, SF, C, H, W)
    return out.reshape(N, SF, 2, H, W)


# speedup vs baseline: 289.9521x; 1.2459x over previous
import functools
import math

import jax
import jax.numpy as jnp
from jax.experimental import pallas as pl
from jax.experimental.pallas import tpu as pltpu

_R = 3
_BW = 0.1

_DELTAS = [(0, 1), (0, 2),
           (1, -2), (1, -1), (1, 0), (1, 1), (1, 2),
           (2, -2), (2, -1), (2, 0), (2, 1), (2, 2)]


def _entropy_kernel(x_ref, xn_ref, out_ref, *, H, W, C):
    Ho, Wo = H - _R + 1, W - _R + 1
    inv2h2 = 1.0 / (2.0 * _BW * _BW)
    M = _R * _R
    scale = jnp.float32(math.sqrt(inv2h2 / math.log(2.0)))

    def build_maps(ref):
        Xs = [ref[0, c] * scale for c in range(C)]
        Xd = {0: Xs}
        for dj in (-2, -1, 1, 2):
            Xd[dj] = [jnp.roll(xc, -dj, axis=1) for xc in Xs]
        Xr = {0: Xs}
        for di in (1, 2):
            Xr[di] = [jnp.roll(xc, di, axis=0) for xc in Xs]
        Ft = {}
        for (di, dj) in _DELTAS:
            acc = None
            for c in range(C):
                d = Xr[di][c] - Xd[dj][c]
                acc = d * d if acc is None else acc + d * d
            Ft[(di, dj)] = jnp.exp2(-acc)
        return Ft

    Ftf = build_maps(x_ref)
    Ftn = build_maps(xn_ref)
    Gt = {d: (Ftf[d] * Ftn[d]).astype(jnp.bfloat16) for d in _DELTAS}
    Ftf = {d: Ftf[d].astype(jnp.bfloat16) for d in _DELTAS}

    def point_sums(T):
        Cw = {}
        m01, m02 = T[(0, 1)], T[(0, 2)]
        m0m1 = jnp.roll(m01, 1, axis=1)
        m0m2 = jnp.roll(m02, 2, axis=1)
        Cw[(0, -2)] = (m0m2 + m0m1) + 1.0
        Cw[(0, -1)] = (m0m1 + m01) + 1.0
        Cw[(0, 0)] = (m01 + m02) + 1.0
        for di in (1, 2):
            TL = {0: T[(di, 0)]}
            for cp in (-2, -1, 1, 2):
                TL[cp] = jnp.roll(T[(di, cp)], cp, axis=1)
            for cl in (-2, -1, 0):
                Cw[(-di, cl)] = TL[-cl] + TL[-(cl + 1)] + TL[-(cl + 2)]
                Cw[(di, cl)] = T[(di, cl)] + T[(di, cl + 1)] + T[(di, cl + 2)]
        return Cw

    def entropy(T, const):
        Cw = point_sums(T)
        for di in (1, 2):
            for cl in (-2, -1, 0):
                Cw[(di, cl)] = jnp.roll(Cw[(di, cl)], -di, axis=0)
        p = None
        for aj in range(_R):
            cl = -aj
            u = Cw[(-1, cl)] + Cw[(0, cl)]
            v = Cw[(0, cl)] + Cw[(1, cl)]
            s0 = v + Cw[(2, cl)]
            s1 = u + Cw[(1, cl)]
            s2 = Cw[(-2, cl)] + u
            q = s0[0:Ho, :] * s1[1:1 + Ho, :] * s2[2:2 + Ho, :]
            qs = q[:, aj:aj + Wo]
            p = qs if p is None else p * qs
        p = p.astype(jnp.float32)
        return jnp.float32(const) - jnp.log2(p) * jnp.float32(math.log(2.0) / M)

    log_norm = math.log(2.0 * math.pi * _BW * _BW)
    h_m = entropy(Ftf, math.log(float(M)) + 0.5 * C * log_norm)
    h_j = entropy(Gt, math.log(float(M)) + C * log_norm)

    pad = _R // 2
    out_ref[0, :, :, :] = jnp.zeros((2, H, W), jnp.float32)
    out_ref[0, 0, pad:pad + Ho, pad:pad + Wo] = h_m
    out_ref[0, 1, pad:pad + Ho, pad:pad + Wo] = h_j


def _run(x, N, SF, C, H, W, interpret=False):
    spec_x = pl.BlockSpec((1, C, H, W), lambda i: (i, 0, 0, 0))
    spec_xn = pl.BlockSpec(
        (1, C, H, W),
        lambda i, SF=SF: (jnp.where(i % SF == SF - 1, i, i + 1), 0, 0, 0))
    return pl.pallas_call(
        functools.partial(_entropy_kernel, H=H, W=W, C=C),
        grid=(N * SF,),
        in_specs=[spec_x, spec_xn],
        out_specs=pl.BlockSpec((1, 2, H, W), lambda i: (i, 0, 0, 0)),
        out_shape=jax.ShapeDtypeStruct((N * SF, 2, H, W), jnp.float32),
        compiler_params=pltpu.CompilerParams(
            dimension_semantics=("parallel",),
            vmem_limit_bytes=100 * 1024 * 1024,
        ),
        interpret=interpret,
    )(x, x)


def kernel(input):
    N, SF, C, H, W = input.shape
    x = input.reshape(N * SF, C, H, W)
    out = _run(x, N, SF, C, H, W)
    return out.reshape(N, SF, 2, H, W)


# speedup vs baseline: 293.5392x; 1.0124x over previous
import functools
import math

import jax
import jax.numpy as jnp
from jax.experimental import pallas as pl
from jax.experimental.pallas import tpu as pltpu

_R = 3
_BW = 0.1

_DELTAS = [(0, 1), (0, 2),
           (1, -2), (1, -1), (1, 0), (1, 1), (1, 2),
           (2, -2), (2, -1), (2, 0), (2, 1), (2, 2)]


def _entropy_kernel(x_ref, xn_ref, out_ref, *, H, W, C):
    Ho, Wo = H - _R + 1, W - _R + 1
    inv2h2 = 1.0 / (2.0 * _BW * _BW)
    M = _R * _R
    scale = jnp.float32(math.sqrt(inv2h2 / math.log(2.0)))

    def build_maps(ref):
        Xs = [ref[0, c] * scale for c in range(C)]
        Xd = {0: Xs}
        for dj in (-2, -1, 1, 2):
            Xd[dj] = [jnp.roll(xc, -dj, axis=1) for xc in Xs]
        Xr = {0: Xs}
        for di in (1, 2):
            Xr[di] = [jnp.roll(xc, di, axis=0) for xc in Xs]
        Ft = {}
        for (di, dj) in _DELTAS:
            acc = None
            for c in range(C):
                d = Xr[di][c] - Xd[dj][c]
                acc = d * d if acc is None else acc + d * d
            Ft[(di, dj)] = jnp.exp2(-acc).astype(jnp.bfloat16)
        return Ft

    Ftf = build_maps(x_ref)
    Ftn = build_maps(xn_ref)
    Gt = {d: Ftf[d] * Ftn[d] for d in _DELTAS}

    def point_sums(T):
        Cw = {}
        m01, m02 = T[(0, 1)], T[(0, 2)]
        m0m1 = jnp.roll(m01, 1, axis=1)
        m0m2 = jnp.roll(m02, 2, axis=1)
        Cw[(0, -2)] = (m0m2 + m0m1) + 1.0
        Cw[(0, -1)] = (m0m1 + m01) + 1.0
        Cw[(0, 0)] = (m01 + m02) + 1.0
        for di in (1, 2):
            TL = {0: T[(di, 0)]}
            for cp in (-2, -1, 1, 2):
                TL[cp] = jnp.roll(T[(di, cp)], cp, axis=1)
            for cl in (-2, -1, 0):
                Cw[(-di, cl)] = TL[-cl] + TL[-(cl + 1)] + TL[-(cl + 2)]
                Cw[(di, cl)] = T[(di, cl)] + T[(di, cl + 1)] + T[(di, cl + 2)]
        return Cw

    def entropy(T, const):
        Cw = point_sums(T)
        for di in (1, 2):
            for cl in (-2, -1, 0):
                Cw[(di, cl)] = jnp.roll(Cw[(di, cl)], -di, axis=0)
        p = None
        for aj in range(_R):
            cl = -aj
            u = Cw[(-1, cl)] + Cw[(0, cl)]
            v = Cw[(0, cl)] + Cw[(1, cl)]
            s0 = v + Cw[(2, cl)]
            s1 = u + Cw[(1, cl)]
            s2 = Cw[(-2, cl)] + u
            q = s0[0:Ho, :] * s1[1:1 + Ho, :] * s2[2:2 + Ho, :]
            qs = q[:, aj:aj + Wo]
            p = qs if p is None else p * qs
        p = p.astype(jnp.float32)
        return jnp.float32(const) - jnp.log2(p) * jnp.float32(math.log(2.0) / M)

    log_norm = math.log(2.0 * math.pi * _BW * _BW)
    h_m = entropy(Ftf, math.log(float(M)) + 0.5 * C * log_norm)
    h_j = entropy(Gt, math.log(float(M)) + C * log_norm)

    pad = _R // 2
    out_ref[0, :, :, :] = jnp.zeros((2, H, W), jnp.float32)
    out_ref[0, 0, pad:pad + Ho, pad:pad + Wo] = h_m
    out_ref[0, 1, pad:pad + Ho, pad:pad + Wo] = h_j


def _run(x, N, SF, C, H, W, interpret=False):
    spec_x = pl.BlockSpec((1, C, H, W), lambda i: (i, 0, 0, 0))
    spec_xn = pl.BlockSpec(
        (1, C, H, W),
        lambda i, SF=SF: (jnp.where(i % SF == SF - 1, i, i + 1), 0, 0, 0))
    return pl.pallas_call(
        functools.partial(_entropy_kernel, H=H, W=W, C=C),
        grid=(N * SF,),
        in_specs=[spec_x, spec_xn],
        out_specs=pl.BlockSpec((1, 2, H, W), lambda i: (i, 0, 0, 0)),
        out_shape=jax.ShapeDtypeStruct((N * SF, 2, H, W), jnp.float32),
        compiler_params=pltpu.CompilerParams(
            dimension_semantics=("parallel",),
            vmem_limit_bytes=100 * 1024 * 1024,
        ),
        interpret=interpret,
    )(x, x)


def kernel(input):
    N, SF, C, H, W = input.shape
    x = input.reshape(N * SF, C, H, W)
    out = _run(x, N, SF, C, H, W)
    return out.reshape(N, SF, 2, H, W)
